# Initial kernel scaffold; baseline (speedup 1.0000x reference)
import jax
import jax.numpy as jnp
from jax.experimental import pallas as pl


def kernel(x, edge_index, labels, W1, b1, Wc1, bc1, prelu_w, Wc2, bc2):
    raise NotImplementedError("write your pallas kernel here")


# speedup vs baseline: 6.0414x; 6.0414x over previous
import functools

import jax
import jax.numpy as jnp
from jax import lax
from jax.experimental import pallas as pl
from jax.experimental.pallas import tpu as pltpu
from jax.experimental.pallas import tpu_sc as plsc

NC, NS = 2, 16
NW = NC * NS
CHUNK = 80
LANES = 16


def _sc_segsum(x, src, dst, zeros):
  n, d = x.shape
  e = src.shape[0]
  n_chunks = e // CHUNK
  rpt = (n // NS) // 8 * 8
  tail = n - (NS - 1) * rpt

  mesh = plsc.VectorSubcoreMesh(core_axis_name="c", subcore_axis_name="s")

  @functools.partial(
      pl.kernel,
      out_type=(jax.ShapeDtypeStruct((NC * n, d), jnp.float32),
                jax.ShapeDtypeStruct((NW, n), jnp.float32)),
      mesh=mesh,
      compiler_params=pltpu.CompilerParams(needs_layout_passes=False),
      scratch_types=[
          pltpu.VMEM_SHARED((n, d), jnp.float32),
          pltpu.VMEM((CHUNK,), jnp.int32),
          pltpu.VMEM((CHUNK,), jnp.int32),
          pltpu.VMEM((CHUNK, d), jnp.float32),
          pltpu.VMEM((n,), jnp.float32),
          pltpu.SemaphoreType.DMA,
      ],
  )
  def seg_kernel(x_hbm, src_hbm, dst_hbm, zeros_hbm, out_hbm, deg_hbm,
                 acc, src_v, dst_v, rows_v, deg_v, sem):
    cid = lax.axis_index("c")
    sid = lax.axis_index("s")
    wid = cid * NS + sid

    r0 = sid * rpt
    pltpu.sync_copy(zeros_hbm.at[pl.ds(r0, rpt)], acc.at[pl.ds(r0, rpt)])

    @pl.when(sid == NS - 1)
    def _zero_tail():
      t0 = NS * rpt
      pltpu.sync_copy(zeros_hbm.at[pl.ds(t0, tail - rpt)],
                      acc.at[pl.ds(t0, tail - rpt)])

    def zbody(i, _):
      deg_v[pl.ds(i * LANES, LANES)] = jnp.zeros((LANES,), jnp.float32)
      return _

    lax.fori_loop(0, n // LANES, zbody, 0, unroll=False)
    plsc.subcore_barrier()

    def body(i, _):
      c = i * NW + wid
      base = c * CHUNK
      pltpu.sync_copy(src_hbm.at[pl.ds(base, CHUNK)], src_v)
      pltpu.sync_copy(dst_hbm.at[pl.ds(base, CHUNK)], dst_v)
      pltpu.async_copy(x_hbm.at[src_v], rows_v, sem).wait()
      pltpu.sync_copy(rows_v, acc.at[dst_v], add=True)
      for j in range(CHUNK // LANES):
        dvec = dst_v[pl.ds(j * LANES, LANES)]
        cnt, last = plsc.scan_count(dvec)
        plsc.addupdate_scatter(deg_v, [dvec], cnt.astype(jnp.float32),
                               mask=last)
      return _

    lax.fori_loop(0, n_chunks // NW, body, 0, unroll=False)
    plsc.subcore_barrier()

    pltpu.sync_copy(acc.at[pl.ds(r0, rpt)],
                    out_hbm.at[pl.ds(cid * n + r0, rpt)])

    @pl.when(sid == NS - 1)
    def _write_tail():
      t0 = NS * rpt
      pltpu.sync_copy(acc.at[pl.ds(t0, tail - rpt)],
                      out_hbm.at[pl.ds(cid * n + t0, tail - rpt)])

    pltpu.sync_copy(deg_v, deg_hbm.at[wid])

  return seg_kernel(x, src, dst, zeros)


def _tc_mlp_body(x_ref, a0_ref, a1_ref, deg_ref, w1a_ref, w1b_ref, b1_ref,
                 wc1_ref, bc1_ref, pw_ref, wc2_ref, bc2_ref, out_ref):
  a = a0_ref[...] + a1_ref[...]
  deg = jnp.sum(deg_ref[...], axis=1)[:, None]
  agg = a / jnp.maximum(deg, 1.0)
  h = x_ref[...] @ w1a_ref[...] + agg @ w1b_ref[...] + b1_ref[...][None, :]
  h = jnp.maximum(h, 0.0)
  p1 = h @ wc1_ref[...] + bc1_ref[...][None, :]
  p1 = jnp.where(p1 >= 0, p1, pw_ref[...][None, :] * p1)
  out_ref[...] = p1 @ wc2_ref[...] + bc2_ref[...][None, :]


def _tc_mlp(x, partials, degs, w1a, w1b, b1, wc1, bc1, prelu_w, wc2, bc2):
  n, d = x.shape
  h = wc1.shape[0]
  c = wc2.shape[1]
  bn = 1000
  grid = n // bn

  full = lambda shape: pl.BlockSpec(shape, lambda i: (0,) * len(shape))
  return pl.pallas_call(
      _tc_mlp_body,
      grid=(grid,),
      in_specs=[
          pl.BlockSpec((bn, d), lambda i: (i, 0)),
          pl.BlockSpec((bn, d), lambda i: (i, 0)),
          pl.BlockSpec((bn, d), lambda i: (i + n // bn, 0)),
          pl.BlockSpec((bn, NW), lambda i: (i, 0)),
          full((d, h)), full((d, h)), full((h,)),
          full((h, h)), full((h,)), full((h,)),
          full((h, c)), full((c,)),
      ],
      out_specs=pl.BlockSpec((bn, c), lambda i: (i, 0)),
      out_shape=jax.ShapeDtypeStruct((n, c), jnp.float32),
  )(x, partials, partials, degs, w1a, w1b, b1, wc1, bc1, prelu_w, wc2, bc2)


def kernel(x, edge_index, labels, W1, b1, Wc1, bc1, prelu_w, Wc2, bc2):
  n, d = x.shape
  zeros = jnp.zeros((n, d), jnp.float32)
  partials, degs = _sc_segsum(x, edge_index[0], edge_index[1], zeros)
  return _tc_mlp(x, partials, degs.T, W1[:d], W1[d:], b1,
                 Wc1, bc1, prelu_w, Wc2, bc2)
